# baseline (speedup 1.0000x reference)
import functools

import jax
import jax.numpy as jnp
from jax import lax
from jax.experimental import pallas as pl
from jax.experimental.pallas import tpu as pltpu
from jax.experimental.pallas import tpu_sc as plsc

N = 10000
E = 320000
D_NODE = 128
D_EDGE = 16
D_GLOBAL = 32
LATENT = 32

_NC = 2
_NS = 16
_NW = _NC * _NS
_PER_TILE = E // _NW
_WIN = 80
_NWIN = _PER_TILE // _WIN

_N_BLK = 2000
_EA_BLK = 16000


def _tables_body(x_ref, w_ref, xr_ref, xc_ref, xn_ref):
    t = jnp.dot(x_ref[...], w_ref[...], preferred_element_type=jnp.float32)
    xr_ref[...] = t[:, 0:LATENT]
    xc_ref[...] = t[:, LATENT:2 * LATENT]
    xn_ref[...] = t[:, 2 * LATENT:3 * LATENT]


def _tables(x, w_cat):
    return pl.pallas_call(
        _tables_body,
        grid=(N // _N_BLK,),
        in_specs=[
            pl.BlockSpec((_N_BLK, D_NODE), lambda i: (i, 0)),
            pl.BlockSpec((D_NODE, 3 * LATENT), lambda i: (0, 0)),
        ],
        out_specs=[
            pl.BlockSpec((_N_BLK, LATENT), lambda i: (i, 0)),
            pl.BlockSpec((_N_BLK, LATENT), lambda i: (i, 0)),
            pl.BlockSpec((_N_BLK, LATENT), lambda i: (i, 0)),
        ],
        out_shape=[jax.ShapeDtypeStruct((N, LATENT), jnp.float32)] * 3,
    )(x, w_cat)


def _ea_body(ea0_ref, ea1_ref, ea2_ref, ea3_ref, w64_ref, u_ref, wu_ref,
             b_ref, ea_ref):
    const = jnp.dot(u_ref[...], wu_ref[...], preferred_element_type=jnp.float32) + b_ref[...]
    cat = jnp.concatenate(
        [ea0_ref[...], ea1_ref[...], ea2_ref[...], ea3_ref[...]], axis=0)
    t = lax.dot_general(cat, w64_ref[...], (((0,), (0,)), ((), ())),
                        preferred_element_type=jnp.float32)
    ea_ref[...] = t + jnp.concatenate([const] * 4, axis=1)


def _ea_stage(edge_attr_T, we_e64, u, we_u, be1):
    nblk = (E // 4) // _EA_BLK
    specs = [
        pl.BlockSpec((D_EDGE, _EA_BLK), lambda i, k=k: (0, k * nblk + i))
        for k in range(4)
    ]
    return pl.pallas_call(
        _ea_body,
        grid=(nblk,),
        in_specs=specs + [
            pl.BlockSpec((4 * D_EDGE, 128), lambda i: (0, 0)),
            pl.BlockSpec((1, D_GLOBAL), lambda i: (0, 0)),
            pl.BlockSpec((D_GLOBAL, LATENT), lambda i: (0, 0)),
            pl.BlockSpec((1, LATENT), lambda i: (0, 0)),
        ],
        out_specs=pl.BlockSpec((_EA_BLK, 128), lambda i: (i, 0)),
        out_shape=jax.ShapeDtypeStruct((E // 4, 128), jnp.float32),
    )(edge_attr_T, edge_attr_T, edge_attr_T, edge_attr_T, we_e64, u, we_u, be1)


def _perm_body(r_ref, c_ref, p_ref, rp_ref, cp_ref):
    pmat = p_ref[...]
    for src, dst in ((r_ref, rp_ref), (c_ref, cp_ref)):
        cat = jnp.concatenate(
            [src[625 * q:625 * (q + 1), :].astype(jnp.float32)
             for q in range(4)], axis=1)
        dst[...] = lax.dot_general(
            cat, pmat, (((1,), (0,)), ((), ())),
            preferred_element_type=jnp.float32,
            precision=lax.Precision.HIGHEST).astype(jnp.int32)


def _perm_stage(row128, col128, pmat):
    return pl.pallas_call(
        _perm_body,
        out_shape=[jax.ShapeDtypeStruct((E // 512, 512), jnp.int32)] * 2,
    )(row128, col128, pmat)


_ROWS_W = _WIN // 4
_SUBW = 5
_SW = _SUBW * _WIN


def _sc_edge_body(xr_hbm, xc_hbm, ea_hbm, rowp_hbm, colp_hbm,
                  z32_hbm, z16_hbm,
                  h_hbm, acc_hbm, cnt_hbm,
                  gr, gc, ones, acc_s, cnt_s, semr, semc, semw):
    cid = lax.axis_index("c")
    sid = lax.axis_index("s")

    @pl.loop(0, _WIN)
    def _(i):
        ones[i, :] = jnp.full((16,), 1.0, dtype=jnp.float32)

    @pl.when(sid == 0)
    def _():
        pltpu.sync_copy(z32_hbm, acc_s)
        pltpu.sync_copy(z16_hbm, cnt_s)

    plsc.subcore_barrier()

    def window(idxr_v, idxc_v, ea_v, h_v):
        cps = []
        for k in range(_SUBW):
            cps.append(pltpu.async_copy(
                xr_hbm.at[idxr_v.at[k]],
                gr.at[pl.ds(_WIN * k, _WIN)], semr))
            cps.append(pltpu.async_copy(
                xc_hbm.at[idxc_v.at[k]],
                gc.at[pl.ds(_WIN * k, _WIN)], semc))
        for cp in cps:
            cp.wait()

        @pl.loop(0, _SW)
        def _(j):
            for half in range(2):
                s = pl.ds(half * 16, 16)
                v = gr[j, s] + gc[j, s] + ea_v[j, s]
                h_v[j, s] = jnp.maximum(v, 0.0)

        wps = []
        for k in range(_SUBW):
            wps.append(pltpu.async_copy(h_v.at[pl.ds(_WIN * k, _WIN)],
                                        acc_s.at[idxc_v.at[k]], semw,
                                        add=True))
            wps.append(pltpu.async_copy(ones, cnt_s.at[idxc_v.at[k]], semw,
                                        add=True))
        for wp in wps:
            wp.wait()

    pltpu.emit_pipeline(
        window,
        grid=(E // _SW,),
        in_specs=[
            pl.BlockSpec((_SUBW, _WIN), lambda i: (i, 0)),
            pl.BlockSpec((_SUBW, _WIN), lambda i: (i, 0)),
            pl.BlockSpec((_SW, LATENT), lambda i: (i, 0)),
        ],
        out_specs=[pl.BlockSpec((_SW, LATENT), lambda i: (i, 0))],
        core_axis_name=("c", "s"),
        dimension_semantics=(pltpu.PARALLEL,),
    )(rowp_hbm, colp_hbm, ea_hbm, h_hbm)

    plsc.subcore_barrier()

    @pl.when(sid == 0)
    def _():
        pltpu.sync_copy(acc_s, acc_hbm.at[cid])
        pltpu.sync_copy(cnt_s, cnt_hbm.at[cid])


def _sc_edge_stage(xr, xc, ea, row_p, col_p, z32, z16):
    fn = pl.kernel(
        _sc_edge_body,
        out_type=[
            jax.ShapeDtypeStruct((E, LATENT), jnp.float32),
            jax.ShapeDtypeStruct((_NC, N, LATENT), jnp.float32),
            jax.ShapeDtypeStruct((_NC, N, 16), jnp.float32),
        ],
        mesh=plsc.VectorSubcoreMesh(core_axis_name="c", subcore_axis_name="s"),
        compiler_params=pltpu.CompilerParams(use_tc_tiling_on_sc=False),
        scratch_types=[
            pltpu.VMEM((_SW, LATENT), jnp.float32),
            pltpu.VMEM((_SW, LATENT), jnp.float32),
            pltpu.VMEM((_WIN, 16), jnp.float32),
            pltpu.VMEM_SHARED((N, LATENT), jnp.float32),
            pltpu.VMEM_SHARED((N, 16), jnp.float32),
            pltpu.SemaphoreType.DMA,
            pltpu.SemaphoreType.DMA,
            pltpu.SemaphoreType.DMA,
        ],
    )
    return fn(xr, xc, ea, row_p.reshape(E // _WIN, _WIN),
              col_p.reshape(E // _WIN, _WIN), z32, z16)


def _edges_body(h_ref, w_ref, bT_ref, e2T_ref, sum_ref):
    i = pl.program_id(0)
    q = pl.program_id(1)
    h4 = h_ref[...]

    def mk(qq):
        def f():
            hq = h4[:, 32 * qq:32 * (qq + 1)]
            return lax.dot_general(
                w_ref[...], hq, (((0,), (1,)), ((), ())),
                preferred_element_type=jnp.float32) + bT_ref[...]
        return f

    e2T_ref[...] = lax.switch(q, [mk(0), mk(1), mk(2), mk(3)])

    @pl.when(jnp.logical_and(i == 0, q == 0))
    def _():
        sum_ref[...] = jnp.sum(h4, axis=0, keepdims=True)

    @pl.when(jnp.logical_and(i > 0, q == 0))
    def _():
        sum_ref[...] += jnp.sum(h4, axis=0, keepdims=True)


def _edges_stage(h4, we2, be2T):
    nblk = (E // 4) // _EA_BLK
    return pl.pallas_call(
        _edges_body,
        grid=(nblk, 4),
        in_specs=[
            pl.BlockSpec((_EA_BLK, 128), lambda i, q: (i, 0)),
            pl.BlockSpec((LATENT, LATENT), lambda i, q: (0, 0)),
            pl.BlockSpec((LATENT, 1), lambda i, q: (0, 0)),
        ],
        out_specs=[
            pl.BlockSpec((LATENT, _EA_BLK), lambda i, q: (0, q * nblk + i)),
            pl.BlockSpec((1, 128), lambda i, q: (0, 0)),
        ],
        out_shape=[
            jax.ShapeDtypeStruct((LATENT, E), jnp.float32),
            jax.ShapeDtypeStruct((1, 128), jnp.float32),
        ],
    )(h4, we2, be2T)


def _nodes_body(acc_ref, cnt_ref, xn_ref, we2_ref, be2_ref, wn1e_ref,
                u_ref, wn1u_ref, bn1_ref, wn2_ref, bn2_ref, x2_ref, sum_ref):
    i = pl.program_id(0)
    sum_h = acc_ref[0] + acc_ref[1]
    cnt_raw = cnt_ref[0, :, 0:1] + cnt_ref[1, :, 0:1]
    cntc = jnp.maximum(cnt_raw, 1.0)
    edge_agg = (
        jnp.dot(sum_h, we2_ref[...], preferred_element_type=jnp.float32)
        + cnt_raw * be2_ref[...]
    ) / cntc
    const = (
        jnp.dot(u_ref[...], wn1u_ref[...], preferred_element_type=jnp.float32)
        + bn1_ref[...]
    )
    pre = (
        xn_ref[...]
        + jnp.dot(edge_agg, wn1e_ref[...], preferred_element_type=jnp.float32)
        + const
    )
    x2T = lax.dot_general(
        wn2_ref[...], jnp.maximum(pre, 0.0), (((0,), (1,)), ((), ())),
        preferred_element_type=jnp.float32) + bn2_ref[...]
    x2_ref[...] = x2T
    s = jnp.sum(x2T, axis=1, keepdims=True)

    @pl.when(i == 0)
    def _():
        sum_ref[...] = s

    @pl.when(i > 0)
    def _():
        sum_ref[...] += s


def _nodes_stage(acc, cnt, xn, we2, be2, wn1_e, u, wn1_u, bn1, wn2, bn2T):
    return pl.pallas_call(
        _nodes_body,
        grid=(1,),
        in_specs=[
            pl.BlockSpec((_NC, N, LATENT), lambda i: (0, 0, 0)),
            pl.BlockSpec((_NC, N, 16), lambda i: (0, 0, 0)),
            pl.BlockSpec((N, LATENT), lambda i: (0, 0)),
            pl.BlockSpec((LATENT, LATENT), lambda i: (0, 0)),
            pl.BlockSpec((1, LATENT), lambda i: (0, 0)),
            pl.BlockSpec((LATENT, LATENT), lambda i: (0, 0)),
            pl.BlockSpec((1, D_GLOBAL), lambda i: (0, 0)),
            pl.BlockSpec((D_GLOBAL, LATENT), lambda i: (0, 0)),
            pl.BlockSpec((1, LATENT), lambda i: (0, 0)),
            pl.BlockSpec((LATENT, LATENT), lambda i: (0, 0)),
            pl.BlockSpec((LATENT, 1), lambda i: (0, 0)),
        ],
        out_specs=[
            pl.BlockSpec((LATENT, N), lambda i: (0, 0)),
            pl.BlockSpec((LATENT, 1), lambda i: (0, 0)),
        ],
        out_shape=[
            jax.ShapeDtypeStruct((LATENT, N), jnp.float32),
            jax.ShapeDtypeStruct((LATENT, 1), jnp.float32),
        ],
    )(acc, cnt, xn, we2, be2, wn1_e, u, wn1_u, bn1, wn2, bn2T)


def _global_body(sx2_ref, sh_ref, u_ref, we2_ref, be2_ref,
                 wg1u_ref, wg1n_ref, wg1e_ref,
                 bg1_ref, wg2_ref, bg2_ref, u2_ref):
    node_aggT = sx2_ref[...] * (1.0 / N)
    s128 = sh_ref[...]
    sum_h = (s128[:, 0:32] + s128[:, 32:64] + s128[:, 64:96] + s128[:, 96:128])
    edge_mean = (
        jnp.dot(sum_h * (1.0 / E), we2_ref[...],
                preferred_element_type=jnp.float32)
        + be2_ref[...]
    )
    pre = (
        jnp.dot(u_ref[...], wg1u_ref[...], preferred_element_type=jnp.float32)
        + lax.dot_general(node_aggT, wg1n_ref[...], (((0,), (0,)), ((), ())),
                          preferred_element_type=jnp.float32)
        + jnp.dot(edge_mean, wg1e_ref[...], preferred_element_type=jnp.float32)
        + bg1_ref[...]
    )
    u2_ref[...] = (
        jnp.dot(jnp.maximum(pre, 0.0), wg2_ref[...],
                preferred_element_type=jnp.float32)
        + bg2_ref[...]
    )


def _global_stage(sum_x2T, sum_h, u, we2, be2, wg1_u, wg1_n, wg1_e, bg1, wg2, bg2):
    return pl.pallas_call(
        _global_body,
        out_shape=jax.ShapeDtypeStruct((1, D_GLOBAL), jnp.float32),
    )(sum_x2T, sum_h, u, we2, be2, wg1_u, wg1_n, wg1_e, bg1, wg2, bg2)


def kernel(x, edge_index, edge_attr, u, We1, be1, We2, be2,
           Wn1, bn1, Wn2, bn2, Wg1, bg1, Wg2, bg2):
    row = edge_index[0]
    col = edge_index[1]

    we1_e = We1[:D_EDGE]
    we1_r = We1[D_EDGE:D_EDGE + D_NODE]
    we1_c = We1[D_EDGE + D_NODE:D_EDGE + 2 * D_NODE]
    we1_u = We1[D_EDGE + 2 * D_NODE:]
    wn1_x = Wn1[:D_NODE]
    wn1_e = Wn1[D_NODE:D_NODE + LATENT]
    wn1_u = Wn1[D_NODE + LATENT:]
    wg1_u = Wg1[:D_GLOBAL]
    wg1_n = Wg1[D_GLOBAL:D_GLOBAL + LATENT]
    wg1_e = Wg1[D_GLOBAL + LATENT:]

    w_cat = jnp.concatenate([we1_r, we1_c, wn1_x], axis=1)
    we_e64 = jnp.kron(jnp.eye(4, dtype=jnp.float32), we1_e)
    tt = jnp.arange(512)
    pmat = jax.nn.one_hot(128 * (tt % 4) + tt // 4, 512, axis=0,
                          dtype=jnp.float32)

    xr, xc, xn = _tables(x, w_cat)
    ea4 = _ea_stage(edge_attr.T, we_e64, u, we1_u, be1[None, :])
    rp512, cp512 = _perm_stage(row.reshape(E // 128, 128),
                               col.reshape(E // 128, 128), pmat)
    row_p = rp512.reshape(E)
    col_p = cp512.reshape(E)

    z32 = jnp.zeros((N, LATENT), jnp.float32)
    z16 = jnp.zeros((N, 16), jnp.float32)
    h, acc, cnt = _sc_edge_stage(xr, xc, ea4.reshape(E, LATENT),
                                 row_p, col_p, z32, z16)

    h4 = h.reshape(E // 4, 128)
    e2T, s128 = _edges_stage(h4, We2, be2[:, None])
    x2T, sum_x2T = _nodes_stage(acc, cnt, xn, We2, be2[None, :], wn1_e,
                                u, wn1_u, bn1[None, :], Wn2, bn2[:, None])
    u2 = _global_stage(sum_x2T, s128, u, We2, be2[None, :],
                       wg1_u, wg1_n, wg1_e, bg1[None, :], Wg2, bg2[None, :])
    return (x2T.T, e2T.T, u2)


# speedup vs baseline: 1.0473x; 1.0473x over previous
import functools

import jax
import jax.numpy as jnp
from jax import lax
from jax.experimental import pallas as pl
from jax.experimental.pallas import tpu as pltpu
from jax.experimental.pallas import tpu_sc as plsc

N = 10000
E = 320000
D_NODE = 128
D_EDGE = 16
D_GLOBAL = 32
LATENT = 32

_NC = 2
_NS = 16
_NW = _NC * _NS
_PER_TILE = E // _NW
_WIN = 80
_NWIN = _PER_TILE // _WIN

_N_BLK = 2000
_EA_BLK = 16000


def _tables_body(x_ref, w_ref, xr_ref, xc_ref, xn_ref):
    t = jnp.dot(x_ref[...], w_ref[...], preferred_element_type=jnp.float32)
    xr_ref[...] = t[:, 0:LATENT]
    xc_ref[...] = t[:, LATENT:2 * LATENT]
    xn_ref[...] = t[:, 2 * LATENT:3 * LATENT]


def _tables(x, w_cat):
    return pl.pallas_call(
        _tables_body,
        grid=(N // _N_BLK,),
        in_specs=[
            pl.BlockSpec((_N_BLK, D_NODE), lambda i: (i, 0)),
            pl.BlockSpec((D_NODE, 3 * LATENT), lambda i: (0, 0)),
        ],
        out_specs=[
            pl.BlockSpec((_N_BLK, LATENT), lambda i: (i, 0)),
            pl.BlockSpec((_N_BLK, LATENT), lambda i: (i, 0)),
            pl.BlockSpec((_N_BLK, LATENT), lambda i: (i, 0)),
        ],
        out_shape=[jax.ShapeDtypeStruct((N, LATENT), jnp.float32)] * 3,
    )(x, w_cat)


def _ea_body(ea0_ref, ea1_ref, ea2_ref, ea3_ref, w64_ref, u_ref, wu_ref,
             b_ref, ea_ref):
    const = jnp.dot(u_ref[...], wu_ref[...], preferred_element_type=jnp.float32) + b_ref[...]
    cat = jnp.concatenate(
        [ea0_ref[...], ea1_ref[...], ea2_ref[...], ea3_ref[...]], axis=0)
    t = lax.dot_general(cat, w64_ref[...], (((0,), (0,)), ((), ())),
                        preferred_element_type=jnp.float32)
    ea_ref[...] = t + jnp.concatenate([const] * 4, axis=1)


def _ea_stage(edge_attr_T, we_e64, u, we_u, be1):
    nblk = (E // 4) // _EA_BLK
    specs = [
        pl.BlockSpec((D_EDGE, _EA_BLK), lambda i, k=k: (0, k * nblk + i))
        for k in range(4)
    ]
    return pl.pallas_call(
        _ea_body,
        grid=(nblk,),
        in_specs=specs + [
            pl.BlockSpec((4 * D_EDGE, 128), lambda i: (0, 0)),
            pl.BlockSpec((1, D_GLOBAL), lambda i: (0, 0)),
            pl.BlockSpec((D_GLOBAL, LATENT), lambda i: (0, 0)),
            pl.BlockSpec((1, LATENT), lambda i: (0, 0)),
        ],
        out_specs=pl.BlockSpec((_EA_BLK, 128), lambda i: (i, 0)),
        out_shape=jax.ShapeDtypeStruct((E // 4, 128), jnp.float32),
    )(edge_attr_T, edge_attr_T, edge_attr_T, edge_attr_T, we_e64, u, we_u, be1)


def _perm_body(r_ref, c_ref, p_ref, rp_ref, cp_ref):
    pmat = p_ref[...]
    for src, dst in ((r_ref, rp_ref), (c_ref, cp_ref)):
        cat = jnp.concatenate(
            [src[625 * q:625 * (q + 1), :].astype(jnp.float32)
             for q in range(4)], axis=1)
        dst[...] = lax.dot_general(
            cat, pmat, (((1,), (0,)), ((), ())),
            preferred_element_type=jnp.float32,
            precision=lax.Precision.HIGHEST).astype(jnp.int32)


def _perm_stage(row128, col128, pmat):
    return pl.pallas_call(
        _perm_body,
        out_shape=[jax.ShapeDtypeStruct((E // 512, 512), jnp.int32)] * 2,
    )(row128, col128, pmat)


_ROWS_W = _WIN // 4
_SUBW = 5
_SW = _SUBW * _WIN


def _sc_edge_body(xr_hbm, xc_hbm, ea_hbm, rowp_hbm, colp_hbm,
                  z32_hbm, z16_hbm,
                  h_hbm, acc_hbm, cnt_hbm,
                  gr, gc, ones, acc_s, cnt_s, semr, semc, semw):
    cid = lax.axis_index("c")
    sid = lax.axis_index("s")

    @pl.loop(0, _WIN)
    def _(i):
        ones[i, :] = jnp.full((16,), 1.0, dtype=jnp.float32)

    @pl.when(sid == 0)
    def _():
        pltpu.sync_copy(z32_hbm, acc_s)
        pltpu.sync_copy(z16_hbm, cnt_s)

    plsc.subcore_barrier()

    def window(idxr_v, idxc_v, ea_v, h_v):
        cps = []
        for k in range(_SUBW):
            cps.append(pltpu.async_copy(
                xr_hbm.at[idxr_v.at[k]],
                gr.at[pl.ds(_WIN * k, _WIN)], semr))
            cps.append(pltpu.async_copy(
                xc_hbm.at[idxc_v.at[k]],
                gc.at[pl.ds(_WIN * k, _WIN)], semc))
        for cp in cps:
            cp.wait()

        @pl.loop(0, _SW, step=4)
        def _(j0):
            for dj in range(4):
                j = j0 + dj
                for half in range(2):
                    s = pl.ds(half * 16, 16)
                    v = gr[j, s] + gc[j, s] + ea_v[j, s]
                    h_v[j, s] = jnp.maximum(v, 0.0)

        wps = []
        for k in range(_SUBW):
            wps.append(pltpu.async_copy(h_v.at[pl.ds(_WIN * k, _WIN)],
                                        acc_s.at[idxc_v.at[k]], semw,
                                        add=True))
            wps.append(pltpu.async_copy(ones, cnt_s.at[idxc_v.at[k]], semw,
                                        add=True))
        for wp in wps:
            wp.wait()

    pltpu.emit_pipeline(
        window,
        grid=(E // _SW,),
        in_specs=[
            pl.BlockSpec((_SUBW, _WIN), lambda i: (i, 0)),
            pl.BlockSpec((_SUBW, _WIN), lambda i: (i, 0)),
            pl.BlockSpec((_SW, LATENT), lambda i: (i, 0)),
        ],
        out_specs=[pl.BlockSpec((_SW, LATENT), lambda i: (i, 0))],
        core_axis_name=("c", "s"),
        dimension_semantics=(pltpu.PARALLEL,),
    )(rowp_hbm, colp_hbm, ea_hbm, h_hbm)

    plsc.subcore_barrier()

    @pl.when(sid == 0)
    def _():
        pltpu.sync_copy(acc_s, acc_hbm.at[cid])
        pltpu.sync_copy(cnt_s, cnt_hbm.at[cid])


def _sc_edge_stage(xr, xc, ea, row_p, col_p, z32, z16):
    fn = pl.kernel(
        _sc_edge_body,
        out_type=[
            jax.ShapeDtypeStruct((E, LATENT), jnp.float32),
            jax.ShapeDtypeStruct((_NC, N, LATENT), jnp.float32),
            jax.ShapeDtypeStruct((_NC, N, 16), jnp.float32),
        ],
        mesh=plsc.VectorSubcoreMesh(core_axis_name="c", subcore_axis_name="s"),
        compiler_params=pltpu.CompilerParams(use_tc_tiling_on_sc=False),
        scratch_types=[
            pltpu.VMEM((_SW, LATENT), jnp.float32),
            pltpu.VMEM((_SW, LATENT), jnp.float32),
            pltpu.VMEM((_WIN, 16), jnp.float32),
            pltpu.VMEM_SHARED((N, LATENT), jnp.float32),
            pltpu.VMEM_SHARED((N, 16), jnp.float32),
            pltpu.SemaphoreType.DMA,
            pltpu.SemaphoreType.DMA,
            pltpu.SemaphoreType.DMA,
        ],
    )
    return fn(xr, xc, ea, row_p.reshape(E // _WIN, _WIN),
              col_p.reshape(E // _WIN, _WIN), z32, z16)


def _edges_body(h_ref, w_ref, bT_ref, e0_ref, e1_ref, e2_ref, e3_ref, sum_ref):
    i = pl.program_id(0)
    h4 = h_ref[...]
    for q, out in enumerate((e0_ref, e1_ref, e2_ref, e3_ref)):
        hq = h4[:, 32 * q:32 * (q + 1)]
        out[...] = lax.dot_general(
            w_ref[...], hq, (((0,), (1,)), ((), ())),
            preferred_element_type=jnp.float32) + bT_ref[...]
    s = jnp.sum(h4, axis=0, keepdims=True)

    @pl.when(i == 0)
    def _():
        sum_ref[...] = s

    @pl.when(i > 0)
    def _():
        sum_ref[...] += s


def _edges_stage(h4, we2, be2T):
    return pl.pallas_call(
        _edges_body,
        grid=((E // 4) // _EA_BLK,),
        in_specs=[
            pl.BlockSpec((_EA_BLK, 128), lambda i: (i, 0)),
            pl.BlockSpec((LATENT, LATENT), lambda i: (0, 0)),
            pl.BlockSpec((LATENT, 1), lambda i: (0, 0)),
        ],
        out_specs=[
            pl.BlockSpec((LATENT, _EA_BLK), lambda i: (0, i)),
            pl.BlockSpec((LATENT, _EA_BLK), lambda i: (0, i)),
            pl.BlockSpec((LATENT, _EA_BLK), lambda i: (0, i)),
            pl.BlockSpec((LATENT, _EA_BLK), lambda i: (0, i)),
            pl.BlockSpec((1, 128), lambda i: (0, 0)),
        ],
        out_shape=[
            jax.ShapeDtypeStruct((LATENT, E // 4), jnp.float32),
            jax.ShapeDtypeStruct((LATENT, E // 4), jnp.float32),
            jax.ShapeDtypeStruct((LATENT, E // 4), jnp.float32),
            jax.ShapeDtypeStruct((LATENT, E // 4), jnp.float32),
            jax.ShapeDtypeStruct((1, 128), jnp.float32),
        ],
    )(h4, we2, be2T)


def _nodes_body(acc_ref, cnt_ref, xn_ref, we2_ref, be2_ref, wn1e_ref,
                u_ref, wn1u_ref, bn1_ref, wn2_ref, bn2_ref, x2_ref, sum_ref):
    i = pl.program_id(0)
    sum_h = acc_ref[0] + acc_ref[1]
    cnt_raw = cnt_ref[0, :, 0:1] + cnt_ref[1, :, 0:1]
    cntc = jnp.maximum(cnt_raw, 1.0)
    edge_agg = (
        jnp.dot(sum_h, we2_ref[...], preferred_element_type=jnp.float32)
        + cnt_raw * be2_ref[...]
    ) / cntc
    const = (
        jnp.dot(u_ref[...], wn1u_ref[...], preferred_element_type=jnp.float32)
        + bn1_ref[...]
    )
    pre = (
        xn_ref[...]
        + jnp.dot(edge_agg, wn1e_ref[...], preferred_element_type=jnp.float32)
        + const
    )
    x2T = lax.dot_general(
        wn2_ref[...], jnp.maximum(pre, 0.0), (((0,), (1,)), ((), ())),
        preferred_element_type=jnp.float32) + bn2_ref[...]
    x2_ref[...] = x2T
    s = jnp.sum(x2T, axis=1, keepdims=True)

    @pl.when(i == 0)
    def _():
        sum_ref[...] = s

    @pl.when(i > 0)
    def _():
        sum_ref[...] += s


def _nodes_stage(acc, cnt, xn, we2, be2, wn1_e, u, wn1_u, bn1, wn2, bn2T):
    return pl.pallas_call(
        _nodes_body,
        grid=(1,),
        in_specs=[
            pl.BlockSpec((_NC, N, LATENT), lambda i: (0, 0, 0)),
            pl.BlockSpec((_NC, N, 16), lambda i: (0, 0, 0)),
            pl.BlockSpec((N, LATENT), lambda i: (0, 0)),
            pl.BlockSpec((LATENT, LATENT), lambda i: (0, 0)),
            pl.BlockSpec((1, LATENT), lambda i: (0, 0)),
            pl.BlockSpec((LATENT, LATENT), lambda i: (0, 0)),
            pl.BlockSpec((1, D_GLOBAL), lambda i: (0, 0)),
            pl.BlockSpec((D_GLOBAL, LATENT), lambda i: (0, 0)),
            pl.BlockSpec((1, LATENT), lambda i: (0, 0)),
            pl.BlockSpec((LATENT, LATENT), lambda i: (0, 0)),
            pl.BlockSpec((LATENT, 1), lambda i: (0, 0)),
        ],
        out_specs=[
            pl.BlockSpec((LATENT, N), lambda i: (0, 0)),
            pl.BlockSpec((LATENT, 1), lambda i: (0, 0)),
        ],
        out_shape=[
            jax.ShapeDtypeStruct((LATENT, N), jnp.float32),
            jax.ShapeDtypeStruct((LATENT, 1), jnp.float32),
        ],
    )(acc, cnt, xn, we2, be2, wn1_e, u, wn1_u, bn1, wn2, bn2T)


def _global_body(sx2_ref, sh_ref, u_ref, we2_ref, be2_ref,
                 wg1u_ref, wg1n_ref, wg1e_ref,
                 bg1_ref, wg2_ref, bg2_ref, u2_ref):
    node_aggT = sx2_ref[...] * (1.0 / N)
    s128 = sh_ref[...]
    sum_h = (s128[:, 0:32] + s128[:, 32:64] + s128[:, 64:96] + s128[:, 96:128])
    edge_mean = (
        jnp.dot(sum_h * (1.0 / E), we2_ref[...],
                preferred_element_type=jnp.float32)
        + be2_ref[...]
    )
    pre = (
        jnp.dot(u_ref[...], wg1u_ref[...], preferred_element_type=jnp.float32)
        + lax.dot_general(node_aggT, wg1n_ref[...], (((0,), (0,)), ((), ())),
                          preferred_element_type=jnp.float32)
        + jnp.dot(edge_mean, wg1e_ref[...], preferred_element_type=jnp.float32)
        + bg1_ref[...]
    )
    u2_ref[...] = (
        jnp.dot(jnp.maximum(pre, 0.0), wg2_ref[...],
                preferred_element_type=jnp.float32)
        + bg2_ref[...]
    )


def _global_stage(sum_x2T, sum_h, u, we2, be2, wg1_u, wg1_n, wg1_e, bg1, wg2, bg2):
    return pl.pallas_call(
        _global_body,
        out_shape=jax.ShapeDtypeStruct((1, D_GLOBAL), jnp.float32),
    )(sum_x2T, sum_h, u, we2, be2, wg1_u, wg1_n, wg1_e, bg1, wg2, bg2)


def kernel(x, edge_index, edge_attr, u, We1, be1, We2, be2,
           Wn1, bn1, Wn2, bn2, Wg1, bg1, Wg2, bg2):
    row = edge_index[0]
    col = edge_index[1]

    we1_e = We1[:D_EDGE]
    we1_r = We1[D_EDGE:D_EDGE + D_NODE]
    we1_c = We1[D_EDGE + D_NODE:D_EDGE + 2 * D_NODE]
    we1_u = We1[D_EDGE + 2 * D_NODE:]
    wn1_x = Wn1[:D_NODE]
    wn1_e = Wn1[D_NODE:D_NODE + LATENT]
    wn1_u = Wn1[D_NODE + LATENT:]
    wg1_u = Wg1[:D_GLOBAL]
    wg1_n = Wg1[D_GLOBAL:D_GLOBAL + LATENT]
    wg1_e = Wg1[D_GLOBAL + LATENT:]

    w_cat = jnp.concatenate([we1_r, we1_c, wn1_x], axis=1)
    we_e64 = jnp.kron(jnp.eye(4, dtype=jnp.float32), we1_e)
    tt = jnp.arange(512)
    pmat = jax.nn.one_hot(128 * (tt % 4) + tt // 4, 512, axis=0,
                          dtype=jnp.float32)

    xr, xc, xn = _tables(x, w_cat)
    ea4 = _ea_stage(edge_attr.T, we_e64, u, we1_u, be1[None, :])
    rp512, cp512 = _perm_stage(row.reshape(E // 128, 128),
                               col.reshape(E // 128, 128), pmat)
    row_p = rp512.reshape(E)
    col_p = cp512.reshape(E)

    z32 = jnp.zeros((N, LATENT), jnp.float32)
    z16 = jnp.zeros((N, 16), jnp.float32)
    h, acc, cnt = _sc_edge_stage(xr, xc, ea4.reshape(E, LATENT),
                                 row_p, col_p, z32, z16)

    h4 = h.reshape(E // 4, 128)
    e2T0, e2T1, e2T2, e2T3, s128 = _edges_stage(h4, We2, be2[:, None])
    e2T = jnp.concatenate([e2T0, e2T1, e2T2, e2T3], axis=1)
    x2T, sum_x2T = _nodes_stage(acc, cnt, xn, We2, be2[None, :], wn1_e,
                                u, wn1_u, bn1[None, :], Wn2, bn2[:, None])
    u2 = _global_stage(sum_x2T, s128, u, We2, be2[None, :],
                       wg1_u, wg1_n, wg1_e, bg1[None, :], Wg2, bg2[None, :])
    return (x2T.T, e2T.T, u2)


# speedup vs baseline: 1.0481x; 1.0008x over previous
import functools

import jax
import jax.numpy as jnp
from jax import lax
from jax.experimental import pallas as pl
from jax.experimental.pallas import tpu as pltpu
from jax.experimental.pallas import tpu_sc as plsc

N = 10000
E = 320000
D_NODE = 128
D_EDGE = 16
D_GLOBAL = 32
LATENT = 32

_NC = 2
_NS = 16
_NW = _NC * _NS
_PER_TILE = E // _NW
_WIN = 80
_NWIN = _PER_TILE // _WIN

_N_BLK = 2000
_EA_BLK = 16000


def _tables_body(x_ref, w_ref, xr_ref, xc_ref, xn_ref):
    t = jnp.dot(x_ref[...], w_ref[...], preferred_element_type=jnp.float32)
    xr_ref[...] = t[:, 0:LATENT]
    xc_ref[...] = t[:, LATENT:2 * LATENT]
    xn_ref[...] = t[:, 2 * LATENT:3 * LATENT]


def _tables(x, w_cat):
    return pl.pallas_call(
        _tables_body,
        grid=(N // _N_BLK,),
        in_specs=[
            pl.BlockSpec((_N_BLK, D_NODE), lambda i: (i, 0)),
            pl.BlockSpec((D_NODE, 3 * LATENT), lambda i: (0, 0)),
        ],
        out_specs=[
            pl.BlockSpec((_N_BLK, LATENT), lambda i: (i, 0)),
            pl.BlockSpec((_N_BLK, LATENT), lambda i: (i, 0)),
            pl.BlockSpec((_N_BLK, LATENT), lambda i: (i, 0)),
        ],
        out_shape=[jax.ShapeDtypeStruct((N, LATENT), jnp.float32)] * 3,
    )(x, w_cat)


def _ea_body(ea0_ref, ea1_ref, ea2_ref, ea3_ref, w64_ref, u_ref, wu_ref,
             b_ref, ea_ref):
    const = jnp.dot(u_ref[...], wu_ref[...], preferred_element_type=jnp.float32) + b_ref[...]
    cat = jnp.concatenate(
        [ea0_ref[...], ea1_ref[...], ea2_ref[...], ea3_ref[...]], axis=0)
    t = lax.dot_general(cat, w64_ref[...], (((0,), (0,)), ((), ())),
                        preferred_element_type=jnp.float32)
    ea_ref[...] = t + jnp.concatenate([const] * 4, axis=1)


def _ea_stage(edge_attr_T, we_e64, u, we_u, be1):
    nblk = (E // 4) // _EA_BLK
    specs = [
        pl.BlockSpec((D_EDGE, _EA_BLK), lambda i, k=k: (0, k * nblk + i))
        for k in range(4)
    ]
    return pl.pallas_call(
        _ea_body,
        grid=(nblk,),
        in_specs=specs + [
            pl.BlockSpec((4 * D_EDGE, 128), lambda i: (0, 0)),
            pl.BlockSpec((1, D_GLOBAL), lambda i: (0, 0)),
            pl.BlockSpec((D_GLOBAL, LATENT), lambda i: (0, 0)),
            pl.BlockSpec((1, LATENT), lambda i: (0, 0)),
        ],
        out_specs=pl.BlockSpec((_EA_BLK, 128), lambda i: (i, 0)),
        out_shape=jax.ShapeDtypeStruct((E // 4, 128), jnp.float32),
    )(edge_attr_T, edge_attr_T, edge_attr_T, edge_attr_T, we_e64, u, we_u, be1)


def _perm_body(r_ref, c_ref, p_ref, rp_ref, cp_ref):
    pmat = p_ref[...]
    for src, dst in ((r_ref, rp_ref), (c_ref, cp_ref)):
        cat = jnp.concatenate(
            [src[625 * q:625 * (q + 1), :].astype(jnp.float32)
             for q in range(4)], axis=1)
        dst[...] = lax.dot_general(
            cat, pmat, (((1,), (0,)), ((), ())),
            preferred_element_type=jnp.float32,
            precision=lax.Precision.HIGHEST).astype(jnp.int32)


def _perm_stage(row128, col128, pmat):
    return pl.pallas_call(
        _perm_body,
        out_shape=[jax.ShapeDtypeStruct((E // 512, 512), jnp.int32)] * 2,
    )(row128, col128, pmat)


_ROWS_W = _WIN // 4
_SUBW = 5
_SW = _SUBW * _WIN


def _sc_edge_body(xr_hbm, xc_hbm, ea_hbm, rowp_hbm, colp_hbm,
                  z32_hbm, z16_hbm,
                  h_hbm, acc_hbm, cnt_hbm,
                  gr, gc, ones, acc_s, cnt_s, semr, semc, semw):
    cid = lax.axis_index("c")
    sid = lax.axis_index("s")

    @pl.loop(0, _WIN)
    def _(i):
        ones[i, :] = jnp.full((16,), 1.0, dtype=jnp.float32)

    @pl.when(sid == 0)
    def _():
        pltpu.sync_copy(z32_hbm, acc_s)
        pltpu.sync_copy(z16_hbm, cnt_s)

    plsc.subcore_barrier()

    def window(idxr_v, idxc_v, ea_v, h_v):
        cps = []
        for k in range(_SUBW):
            cps.append(pltpu.async_copy(
                xr_hbm.at[idxr_v.at[k]],
                gr.at[pl.ds(_WIN * k, _WIN)], semr))
            cps.append(pltpu.async_copy(
                xc_hbm.at[idxc_v.at[k]],
                gc.at[pl.ds(_WIN * k, _WIN)], semc))
        for cp in cps:
            cp.wait()

        @pl.loop(0, _SW, step=8)
        def _(j0):
            for dj in range(8):
                j = j0 + dj
                for half in range(2):
                    s = pl.ds(half * 16, 16)
                    v = gr[j, s] + gc[j, s] + ea_v[j, s]
                    h_v[j, s] = jnp.maximum(v, 0.0)

        wps = []
        for k in range(_SUBW):
            wps.append(pltpu.async_copy(h_v.at[pl.ds(_WIN * k, _WIN)],
                                        acc_s.at[idxc_v.at[k]], semw,
                                        add=True))
            wps.append(pltpu.async_copy(ones, cnt_s.at[idxc_v.at[k]], semw,
                                        add=True))
        for wp in wps:
            wp.wait()

    pltpu.emit_pipeline(
        window,
        grid=(E // _SW,),
        in_specs=[
            pl.BlockSpec((_SUBW, _WIN), lambda i: (i, 0)),
            pl.BlockSpec((_SUBW, _WIN), lambda i: (i, 0)),
            pl.BlockSpec((_SW, LATENT), lambda i: (i, 0)),
        ],
        out_specs=[pl.BlockSpec((_SW, LATENT), lambda i: (i, 0))],
        core_axis_name=("c", "s"),
        dimension_semantics=(pltpu.PARALLEL,),
    )(rowp_hbm, colp_hbm, ea_hbm, h_hbm)

    plsc.subcore_barrier()

    @pl.when(sid == 0)
    def _():
        pltpu.sync_copy(acc_s, acc_hbm.at[cid])
        pltpu.sync_copy(cnt_s, cnt_hbm.at[cid])


def _sc_edge_stage(xr, xc, ea, row_p, col_p, z32, z16):
    fn = pl.kernel(
        _sc_edge_body,
        out_type=[
            jax.ShapeDtypeStruct((E, LATENT), jnp.float32),
            jax.ShapeDtypeStruct((_NC, N, LATENT), jnp.float32),
            jax.ShapeDtypeStruct((_NC, N, 16), jnp.float32),
        ],
        mesh=plsc.VectorSubcoreMesh(core_axis_name="c", subcore_axis_name="s"),
        compiler_params=pltpu.CompilerParams(use_tc_tiling_on_sc=False),
        scratch_types=[
            pltpu.VMEM((_SW, LATENT), jnp.float32),
            pltpu.VMEM((_SW, LATENT), jnp.float32),
            pltpu.VMEM((_WIN, 16), jnp.float32),
            pltpu.VMEM_SHARED((N, LATENT), jnp.float32),
            pltpu.VMEM_SHARED((N, 16), jnp.float32),
            pltpu.SemaphoreType.DMA,
            pltpu.SemaphoreType.DMA,
            pltpu.SemaphoreType.DMA,
        ],
    )
    return fn(xr, xc, ea, row_p.reshape(E // _WIN, _WIN),
              col_p.reshape(E // _WIN, _WIN), z32, z16)


def _edges_body(h_ref, w_ref, bT_ref, e0_ref, e1_ref, e2_ref, e3_ref, sum_ref):
    i = pl.program_id(0)
    h4 = h_ref[...]
    for q, out in enumerate((e0_ref, e1_ref, e2_ref, e3_ref)):
        hq = h4[:, 32 * q:32 * (q + 1)]
        out[...] = lax.dot_general(
            w_ref[...], hq, (((0,), (1,)), ((), ())),
            preferred_element_type=jnp.float32) + bT_ref[...]
    s = jnp.sum(h4, axis=0, keepdims=True)

    @pl.when(i == 0)
    def _():
        sum_ref[...] = s

    @pl.when(i > 0)
    def _():
        sum_ref[...] += s


def _edges_stage(h4, we2, be2T):
    return pl.pallas_call(
        _edges_body,
        grid=((E // 4) // _EA_BLK,),
        in_specs=[
            pl.BlockSpec((_EA_BLK, 128), lambda i: (i, 0)),
            pl.BlockSpec((LATENT, LATENT), lambda i: (0, 0)),
            pl.BlockSpec((LATENT, 1), lambda i: (0, 0)),
        ],
        out_specs=[
            pl.BlockSpec((LATENT, _EA_BLK), lambda i: (0, i)),
            pl.BlockSpec((LATENT, _EA_BLK), lambda i: (0, i)),
            pl.BlockSpec((LATENT, _EA_BLK), lambda i: (0, i)),
            pl.BlockSpec((LATENT, _EA_BLK), lambda i: (0, i)),
            pl.BlockSpec((1, 128), lambda i: (0, 0)),
        ],
        out_shape=[
            jax.ShapeDtypeStruct((LATENT, E // 4), jnp.float32),
            jax.ShapeDtypeStruct((LATENT, E // 4), jnp.float32),
            jax.ShapeDtypeStruct((LATENT, E // 4), jnp.float32),
            jax.ShapeDtypeStruct((LATENT, E // 4), jnp.float32),
            jax.ShapeDtypeStruct((1, 128), jnp.float32),
        ],
    )(h4, we2, be2T)


def _nodes_body(acc_ref, cnt_ref, xn_ref, we2_ref, be2_ref, wn1e_ref,
                u_ref, wn1u_ref, bn1_ref, wn2_ref, bn2_ref, x2_ref, sum_ref):
    i = pl.program_id(0)
    sum_h = acc_ref[0] + acc_ref[1]
    cnt_raw = cnt_ref[0, :, 0:1] + cnt_ref[1, :, 0:1]
    cntc = jnp.maximum(cnt_raw, 1.0)
    edge_agg = (
        jnp.dot(sum_h, we2_ref[...], preferred_element_type=jnp.float32)
        + cnt_raw * be2_ref[...]
    ) / cntc
    const = (
        jnp.dot(u_ref[...], wn1u_ref[...], preferred_element_type=jnp.float32)
        + bn1_ref[...]
    )
    pre = (
        xn_ref[...]
        + jnp.dot(edge_agg, wn1e_ref[...], preferred_element_type=jnp.float32)
        + const
    )
    x2T = lax.dot_general(
        wn2_ref[...], jnp.maximum(pre, 0.0), (((0,), (1,)), ((), ())),
        preferred_element_type=jnp.float32) + bn2_ref[...]
    x2_ref[...] = x2T
    s = jnp.sum(x2T, axis=1, keepdims=True)

    @pl.when(i == 0)
    def _():
        sum_ref[...] = s

    @pl.when(i > 0)
    def _():
        sum_ref[...] += s


def _nodes_stage(acc, cnt, xn, we2, be2, wn1_e, u, wn1_u, bn1, wn2, bn2T):
    return pl.pallas_call(
        _nodes_body,
        grid=(1,),
        in_specs=[
            pl.BlockSpec((_NC, N, LATENT), lambda i: (0, 0, 0)),
            pl.BlockSpec((_NC, N, 16), lambda i: (0, 0, 0)),
            pl.BlockSpec((N, LATENT), lambda i: (0, 0)),
            pl.BlockSpec((LATENT, LATENT), lambda i: (0, 0)),
            pl.BlockSpec((1, LATENT), lambda i: (0, 0)),
            pl.BlockSpec((LATENT, LATENT), lambda i: (0, 0)),
            pl.BlockSpec((1, D_GLOBAL), lambda i: (0, 0)),
            pl.BlockSpec((D_GLOBAL, LATENT), lambda i: (0, 0)),
            pl.BlockSpec((1, LATENT), lambda i: (0, 0)),
            pl.BlockSpec((LATENT, LATENT), lambda i: (0, 0)),
            pl.BlockSpec((LATENT, 1), lambda i: (0, 0)),
        ],
        out_specs=[
            pl.BlockSpec((LATENT, N), lambda i: (0, 0)),
            pl.BlockSpec((LATENT, 1), lambda i: (0, 0)),
        ],
        out_shape=[
            jax.ShapeDtypeStruct((LATENT, N), jnp.float32),
            jax.ShapeDtypeStruct((LATENT, 1), jnp.float32),
        ],
    )(acc, cnt, xn, we2, be2, wn1_e, u, wn1_u, bn1, wn2, bn2T)


def _global_body(sx2_ref, sh_ref, u_ref, we2_ref, be2_ref,
                 wg1u_ref, wg1n_ref, wg1e_ref,
                 bg1_ref, wg2_ref, bg2_ref, u2_ref):
    node_aggT = sx2_ref[...] * (1.0 / N)
    s128 = sh_ref[...]
    sum_h = (s128[:, 0:32] + s128[:, 32:64] + s128[:, 64:96] + s128[:, 96:128])
    edge_mean = (
        jnp.dot(sum_h * (1.0 / E), we2_ref[...],
                preferred_element_type=jnp.float32)
        + be2_ref[...]
    )
    pre = (
        jnp.dot(u_ref[...], wg1u_ref[...], preferred_element_type=jnp.float32)
        + lax.dot_general(node_aggT, wg1n_ref[...], (((0,), (0,)), ((), ())),
                          preferred_element_type=jnp.float32)
        + jnp.dot(edge_mean, wg1e_ref[...], preferred_element_type=jnp.float32)
        + bg1_ref[...]
    )
    u2_ref[...] = (
        jnp.dot(jnp.maximum(pre, 0.0), wg2_ref[...],
                preferred_element_type=jnp.float32)
        + bg2_ref[...]
    )


def _global_stage(sum_x2T, sum_h, u, we2, be2, wg1_u, wg1_n, wg1_e, bg1, wg2, bg2):
    return pl.pallas_call(
        _global_body,
        out_shape=jax.ShapeDtypeStruct((1, D_GLOBAL), jnp.float32),
    )(sum_x2T, sum_h, u, we2, be2, wg1_u, wg1_n, wg1_e, bg1, wg2, bg2)


def kernel(x, edge_index, edge_attr, u, We1, be1, We2, be2,
           Wn1, bn1, Wn2, bn2, Wg1, bg1, Wg2, bg2):
    row = edge_index[0]
    col = edge_index[1]

    we1_e = We1[:D_EDGE]
    we1_r = We1[D_EDGE:D_EDGE + D_NODE]
    we1_c = We1[D_EDGE + D_NODE:D_EDGE + 2 * D_NODE]
    we1_u = We1[D_EDGE + 2 * D_NODE:]
    wn1_x = Wn1[:D_NODE]
    wn1_e = Wn1[D_NODE:D_NODE + LATENT]
    wn1_u = Wn1[D_NODE + LATENT:]
    wg1_u = Wg1[:D_GLOBAL]
    wg1_n = Wg1[D_GLOBAL:D_GLOBAL + LATENT]
    wg1_e = Wg1[D_GLOBAL + LATENT:]

    w_cat = jnp.concatenate([we1_r, we1_c, wn1_x], axis=1)
    we_e64 = jnp.kron(jnp.eye(4, dtype=jnp.float32), we1_e)
    tt = jnp.arange(512)
    pmat = jax.nn.one_hot(128 * (tt % 4) + tt // 4, 512, axis=0,
                          dtype=jnp.float32)

    xr, xc, xn = _tables(x, w_cat)
    ea4 = _ea_stage(edge_attr.T, we_e64, u, we1_u, be1[None, :])
    rp512, cp512 = _perm_stage(row.reshape(E // 128, 128),
                               col.reshape(E // 128, 128), pmat)
    row_p = rp512.reshape(E)
    col_p = cp512.reshape(E)

    z32 = jnp.zeros((N, LATENT), jnp.float32)
    z16 = jnp.zeros((N, 16), jnp.float32)
    h, acc, cnt = _sc_edge_stage(xr, xc, ea4.reshape(E, LATENT),
                                 row_p, col_p, z32, z16)

    h4 = h.reshape(E // 4, 128)
    e2T0, e2T1, e2T2, e2T3, s128 = _edges_stage(h4, We2, be2[:, None])
    e2T = jnp.concatenate([e2T0, e2T1, e2T2, e2T3], axis=1)
    x2T, sum_x2T = _nodes_stage(acc, cnt, xn, We2, be2[None, :], wn1_e,
                                u, wn1_u, bn1[None, :], Wn2, bn2[:, None])
    u2 = _global_stage(sum_x2T, s128, u, We2, be2[None, :],
                       wg1_u, wg1_n, wg1_e, bg1[None, :], Wg2, bg2[None, :])
    return (x2T.T, e2T.T, u2)


# speedup vs baseline: 1.0492x; 1.0010x over previous
import functools

import jax
import jax.numpy as jnp
from jax import lax
from jax.experimental import pallas as pl
from jax.experimental.pallas import tpu as pltpu
from jax.experimental.pallas import tpu_sc as plsc

N = 10000
E = 320000
D_NODE = 128
D_EDGE = 16
D_GLOBAL = 32
LATENT = 32

_NC = 2
_NS = 16
_NW = _NC * _NS
_PER_TILE = E // _NW
_WIN = 80
_NWIN = _PER_TILE // _WIN

_N_BLK = 2000
_EA_BLK = 16000


def _tables_body(x_ref, w_ref, xr_ref, xc_ref, xn_ref):
    t = jnp.dot(x_ref[...], w_ref[...], preferred_element_type=jnp.float32)
    xr_ref[...] = t[:, 0:LATENT]
    xc_ref[...] = t[:, LATENT:2 * LATENT]
    xn_ref[...] = t[:, 2 * LATENT:3 * LATENT]


def _tables(x, w_cat):
    return pl.pallas_call(
        _tables_body,
        grid=(N // _N_BLK,),
        in_specs=[
            pl.BlockSpec((_N_BLK, D_NODE), lambda i: (i, 0)),
            pl.BlockSpec((D_NODE, 3 * LATENT), lambda i: (0, 0)),
        ],
        out_specs=[
            pl.BlockSpec((_N_BLK, LATENT), lambda i: (i, 0)),
            pl.BlockSpec((_N_BLK, LATENT), lambda i: (i, 0)),
            pl.BlockSpec((_N_BLK, LATENT), lambda i: (i, 0)),
        ],
        out_shape=[jax.ShapeDtypeStruct((N, LATENT), jnp.float32)] * 3,
    )(x, w_cat)


def _ea_body(ea0_ref, ea1_ref, ea2_ref, ea3_ref, w64_ref, u_ref, wu_ref,
             b_ref, ea_ref):
    const = jnp.dot(u_ref[...], wu_ref[...], preferred_element_type=jnp.float32) + b_ref[...]
    cat = jnp.concatenate(
        [ea0_ref[...], ea1_ref[...], ea2_ref[...], ea3_ref[...]], axis=0)
    t = lax.dot_general(cat, w64_ref[...], (((0,), (0,)), ((), ())),
                        preferred_element_type=jnp.float32)
    ea_ref[...] = t + jnp.concatenate([const] * 4, axis=1)


def _ea_stage(edge_attr_T, we_e64, u, we_u, be1):
    nblk = (E // 4) // _EA_BLK
    specs = [
        pl.BlockSpec((D_EDGE, _EA_BLK), lambda i, k=k: (0, k * nblk + i))
        for k in range(4)
    ]
    return pl.pallas_call(
        _ea_body,
        grid=(nblk,),
        in_specs=specs + [
            pl.BlockSpec((4 * D_EDGE, 128), lambda i: (0, 0)),
            pl.BlockSpec((1, D_GLOBAL), lambda i: (0, 0)),
            pl.BlockSpec((D_GLOBAL, LATENT), lambda i: (0, 0)),
            pl.BlockSpec((1, LATENT), lambda i: (0, 0)),
        ],
        out_specs=pl.BlockSpec((_EA_BLK, 128), lambda i: (i, 0)),
        out_shape=jax.ShapeDtypeStruct((E // 4, 128), jnp.float32),
    )(edge_attr_T, edge_attr_T, edge_attr_T, edge_attr_T, we_e64, u, we_u, be1)


def _perm_body(r_ref, c_ref, p_ref, rp_ref, cp_ref):
    pmat = p_ref[...]
    for src, dst in ((r_ref, rp_ref), (c_ref, cp_ref)):
        cat = jnp.concatenate(
            [src[625 * q:625 * (q + 1), :].astype(jnp.float32)
             for q in range(4)], axis=1)
        dst[...] = lax.dot_general(
            cat, pmat, (((1,), (0,)), ((), ())),
            preferred_element_type=jnp.float32,
            precision=lax.Precision.HIGHEST).astype(jnp.int32)


def _perm_stage(row128, col128, pmat):
    return pl.pallas_call(
        _perm_body,
        out_shape=[jax.ShapeDtypeStruct((E // 512, 512), jnp.int32)] * 2,
    )(row128, col128, pmat)


_ROWS_W = _WIN // 4
_SUBW = 5
_SW = _SUBW * _WIN


def _sc_edge_body(xr_hbm, xc_hbm, ea_hbm, rowp_hbm, colp_hbm,
                  z32_hbm, z16_hbm,
                  h_hbm, acc_hbm, cnt_hbm,
                  gr, gc, ones, acc_s, cnt_s, semr, semc, semw):
    cid = lax.axis_index("c")
    sid = lax.axis_index("s")

    @pl.loop(0, _WIN)
    def _(i):
        ones[i, :] = jnp.full((16,), 1.0, dtype=jnp.float32)

    nsl = N // _NS
    pltpu.sync_copy(z32_hbm.at[pl.ds(sid * nsl, nsl)],
                    acc_s.at[pl.ds(sid * nsl, nsl)])
    pltpu.sync_copy(z16_hbm.at[pl.ds(sid * nsl, nsl)],
                    cnt_s.at[pl.ds(sid * nsl, nsl)])

    plsc.subcore_barrier()

    def window(idxr_v, idxc_v, ea_v, h_v):
        cps = []
        for k in range(_SUBW):
            cps.append(pltpu.async_copy(
                xr_hbm.at[idxr_v.at[k]],
                gr.at[pl.ds(_WIN * k, _WIN)], semr))
            cps.append(pltpu.async_copy(
                xc_hbm.at[idxc_v.at[k]],
                gc.at[pl.ds(_WIN * k, _WIN)], semc))
        for cp in cps:
            cp.wait()

        @pl.loop(0, _SW, step=8)
        def _(j0):
            for dj in range(8):
                j = j0 + dj
                for half in range(2):
                    s = pl.ds(half * 16, 16)
                    v = gr[j, s] + gc[j, s] + ea_v[j, s]
                    h_v[j, s] = jnp.maximum(v, 0.0)

        wps = []
        for k in range(_SUBW):
            wps.append(pltpu.async_copy(h_v.at[pl.ds(_WIN * k, _WIN)],
                                        acc_s.at[idxc_v.at[k]], semw,
                                        add=True))
            wps.append(pltpu.async_copy(ones, cnt_s.at[idxc_v.at[k]], semw,
                                        add=True))
        for wp in wps:
            wp.wait()

    pltpu.emit_pipeline(
        window,
        grid=(E // _SW,),
        in_specs=[
            pl.BlockSpec((_SUBW, _WIN), lambda i: (i, 0)),
            pl.BlockSpec((_SUBW, _WIN), lambda i: (i, 0)),
            pl.BlockSpec((_SW, LATENT), lambda i: (i, 0)),
        ],
        out_specs=[pl.BlockSpec((_SW, LATENT), lambda i: (i, 0))],
        core_axis_name=("c", "s"),
        dimension_semantics=(pltpu.PARALLEL,),
    )(rowp_hbm, colp_hbm, ea_hbm, h_hbm)

    plsc.subcore_barrier()

    pltpu.sync_copy(acc_s.at[pl.ds(sid * nsl, nsl)],
                    acc_hbm.at[cid, pl.ds(sid * nsl, nsl)])
    pltpu.sync_copy(cnt_s.at[pl.ds(sid * nsl, nsl)],
                    cnt_hbm.at[cid, pl.ds(sid * nsl, nsl)])


def _sc_edge_stage(xr, xc, ea, row_p, col_p, z32, z16):
    fn = pl.kernel(
        _sc_edge_body,
        out_type=[
            jax.ShapeDtypeStruct((E, LATENT), jnp.float32),
            jax.ShapeDtypeStruct((_NC, N, LATENT), jnp.float32),
            jax.ShapeDtypeStruct((_NC, N, 16), jnp.float32),
        ],
        mesh=plsc.VectorSubcoreMesh(core_axis_name="c", subcore_axis_name="s"),
        compiler_params=pltpu.CompilerParams(use_tc_tiling_on_sc=False),
        scratch_types=[
            pltpu.VMEM((_SW, LATENT), jnp.float32),
            pltpu.VMEM((_SW, LATENT), jnp.float32),
            pltpu.VMEM((_WIN, 16), jnp.float32),
            pltpu.VMEM_SHARED((N, LATENT), jnp.float32),
            pltpu.VMEM_SHARED((N, 16), jnp.float32),
            pltpu.SemaphoreType.DMA,
            pltpu.SemaphoreType.DMA,
            pltpu.SemaphoreType.DMA,
        ],
    )
    return fn(xr, xc, ea, row_p.reshape(E // _WIN, _WIN),
              col_p.reshape(E // _WIN, _WIN), z32, z16)


def _edges_body(h_ref, w_ref, bT_ref, e0_ref, e1_ref, e2_ref, e3_ref, sum_ref):
    i = pl.program_id(0)
    h4 = h_ref[...]
    for q, out in enumerate((e0_ref, e1_ref, e2_ref, e3_ref)):
        hq = h4[:, 32 * q:32 * (q + 1)]
        out[...] = lax.dot_general(
            w_ref[...], hq, (((0,), (1,)), ((), ())),
            preferred_element_type=jnp.float32) + bT_ref[...]
    s = jnp.sum(h4, axis=0, keepdims=True)

    @pl.when(i == 0)
    def _():
        sum_ref[...] = s

    @pl.when(i > 0)
    def _():
        sum_ref[...] += s


def _edges_stage(h4, we2, be2T):
    return pl.pallas_call(
        _edges_body,
        grid=((E // 4) // _EA_BLK,),
        in_specs=[
            pl.BlockSpec((_EA_BLK, 128), lambda i: (i, 0)),
            pl.BlockSpec((LATENT, LATENT), lambda i: (0, 0)),
            pl.BlockSpec((LATENT, 1), lambda i: (0, 0)),
        ],
        out_specs=[
            pl.BlockSpec((LATENT, _EA_BLK), lambda i: (0, i)),
            pl.BlockSpec((LATENT, _EA_BLK), lambda i: (0, i)),
            pl.BlockSpec((LATENT, _EA_BLK), lambda i: (0, i)),
            pl.BlockSpec((LATENT, _EA_BLK), lambda i: (0, i)),
            pl.BlockSpec((1, 128), lambda i: (0, 0)),
        ],
        out_shape=[
            jax.ShapeDtypeStruct((LATENT, E // 4), jnp.float32),
            jax.ShapeDtypeStruct((LATENT, E // 4), jnp.float32),
            jax.ShapeDtypeStruct((LATENT, E // 4), jnp.float32),
            jax.ShapeDtypeStruct((LATENT, E // 4), jnp.float32),
            jax.ShapeDtypeStruct((1, 128), jnp.float32),
        ],
    )(h4, we2, be2T)


def _nodes_body(acc_ref, cnt_ref, xn_ref, we2_ref, be2_ref, wn1e_ref,
                u_ref, wn1u_ref, bn1_ref, wn2_ref, bn2_ref, x2_ref, sum_ref):
    i = pl.program_id(0)
    sum_h = acc_ref[0] + acc_ref[1]
    cnt_raw = cnt_ref[0, :, 0:1] + cnt_ref[1, :, 0:1]
    cntc = jnp.maximum(cnt_raw, 1.0)
    edge_agg = (
        jnp.dot(sum_h, we2_ref[...], preferred_element_type=jnp.float32)
        + cnt_raw * be2_ref[...]
    ) / cntc
    const = (
        jnp.dot(u_ref[...], wn1u_ref[...], preferred_element_type=jnp.float32)
        + bn1_ref[...]
    )
    pre = (
        xn_ref[...]
        + jnp.dot(edge_agg, wn1e_ref[...], preferred_element_type=jnp.float32)
        + const
    )
    x2T = lax.dot_general(
        wn2_ref[...], jnp.maximum(pre, 0.0), (((0,), (1,)), ((), ())),
        preferred_element_type=jnp.float32) + bn2_ref[...]
    x2_ref[...] = x2T
    s = jnp.sum(x2T, axis=1, keepdims=True)

    @pl.when(i == 0)
    def _():
        sum_ref[...] = s

    @pl.when(i > 0)
    def _():
        sum_ref[...] += s


def _nodes_stage(acc, cnt, xn, we2, be2, wn1_e, u, wn1_u, bn1, wn2, bn2T):
    return pl.pallas_call(
        _nodes_body,
        grid=(1,),
        in_specs=[
            pl.BlockSpec((_NC, N, LATENT), lambda i: (0, 0, 0)),
            pl.BlockSpec((_NC, N, 16), lambda i: (0, 0, 0)),
            pl.BlockSpec((N, LATENT), lambda i: (0, 0)),
            pl.BlockSpec((LATENT, LATENT), lambda i: (0, 0)),
            pl.BlockSpec((1, LATENT), lambda i: (0, 0)),
            pl.BlockSpec((LATENT, LATENT), lambda i: (0, 0)),
            pl.BlockSpec((1, D_GLOBAL), lambda i: (0, 0)),
            pl.BlockSpec((D_GLOBAL, LATENT), lambda i: (0, 0)),
            pl.BlockSpec((1, LATENT), lambda i: (0, 0)),
            pl.BlockSpec((LATENT, LATENT), lambda i: (0, 0)),
            pl.BlockSpec((LATENT, 1), lambda i: (0, 0)),
        ],
        out_specs=[
            pl.BlockSpec((LATENT, N), lambda i: (0, 0)),
            pl.BlockSpec((LATENT, 1), lambda i: (0, 0)),
        ],
        out_shape=[
            jax.ShapeDtypeStruct((LATENT, N), jnp.float32),
            jax.ShapeDtypeStruct((LATENT, 1), jnp.float32),
        ],
    )(acc, cnt, xn, we2, be2, wn1_e, u, wn1_u, bn1, wn2, bn2T)


def _global_body(sx2_ref, sh_ref, u_ref, we2_ref, be2_ref,
                 wg1u_ref, wg1n_ref, wg1e_ref,
                 bg1_ref, wg2_ref, bg2_ref, u2_ref):
    node_aggT = sx2_ref[...] * (1.0 / N)
    s128 = sh_ref[...]
    sum_h = (s128[:, 0:32] + s128[:, 32:64] + s128[:, 64:96] + s128[:, 96:128])
    edge_mean = (
        jnp.dot(sum_h * (1.0 / E), we2_ref[...],
                preferred_element_type=jnp.float32)
        + be2_ref[...]
    )
    pre = (
        jnp.dot(u_ref[...], wg1u_ref[...], preferred_element_type=jnp.float32)
        + lax.dot_general(node_aggT, wg1n_ref[...], (((0,), (0,)), ((), ())),
                          preferred_element_type=jnp.float32)
        + jnp.dot(edge_mean, wg1e_ref[...], preferred_element_type=jnp.float32)
        + bg1_ref[...]
    )
    u2_ref[...] = (
        jnp.dot(jnp.maximum(pre, 0.0), wg2_ref[...],
                preferred_element_type=jnp.float32)
        + bg2_ref[...]
    )


def _global_stage(sum_x2T, sum_h, u, we2, be2, wg1_u, wg1_n, wg1_e, bg1, wg2, bg2):
    return pl.pallas_call(
        _global_body,
        out_shape=jax.ShapeDtypeStruct((1, D_GLOBAL), jnp.float32),
    )(sum_x2T, sum_h, u, we2, be2, wg1_u, wg1_n, wg1_e, bg1, wg2, bg2)


def kernel(x, edge_index, edge_attr, u, We1, be1, We2, be2,
           Wn1, bn1, Wn2, bn2, Wg1, bg1, Wg2, bg2):
    row = edge_index[0]
    col = edge_index[1]

    we1_e = We1[:D_EDGE]
    we1_r = We1[D_EDGE:D_EDGE + D_NODE]
    we1_c = We1[D_EDGE + D_NODE:D_EDGE + 2 * D_NODE]
    we1_u = We1[D_EDGE + 2 * D_NODE:]
    wn1_x = Wn1[:D_NODE]
    wn1_e = Wn1[D_NODE:D_NODE + LATENT]
    wn1_u = Wn1[D_NODE + LATENT:]
    wg1_u = Wg1[:D_GLOBAL]
    wg1_n = Wg1[D_GLOBAL:D_GLOBAL + LATENT]
    wg1_e = Wg1[D_GLOBAL + LATENT:]

    w_cat = jnp.concatenate([we1_r, we1_c, wn1_x], axis=1)
    we_e64 = jnp.kron(jnp.eye(4, dtype=jnp.float32), we1_e)
    tt = jnp.arange(512)
    pmat = jax.nn.one_hot(128 * (tt % 4) + tt // 4, 512, axis=0,
                          dtype=jnp.float32)

    xr, xc, xn = _tables(x, w_cat)
    ea4 = _ea_stage(edge_attr.T, we_e64, u, we1_u, be1[None, :])
    rp512, cp512 = _perm_stage(row.reshape(E // 128, 128),
                               col.reshape(E // 128, 128), pmat)
    row_p = rp512.reshape(E)
    col_p = cp512.reshape(E)

    z32 = jnp.zeros((N, LATENT), jnp.float32)
    z16 = jnp.zeros((N, 16), jnp.float32)
    h, acc, cnt = _sc_edge_stage(xr, xc, ea4.reshape(E, LATENT),
                                 row_p, col_p, z32, z16)

    h4 = h.reshape(E // 4, 128)
    e2T0, e2T1, e2T2, e2T3, s128 = _edges_stage(h4, We2, be2[:, None])
    e2T = jnp.concatenate([e2T0, e2T1, e2T2, e2T3], axis=1)
    x2T, sum_x2T = _nodes_stage(acc, cnt, xn, We2, be2[None, :], wn1_e,
                                u, wn1_u, bn1[None, :], Wn2, bn2[:, None])
    u2 = _global_stage(sum_x2T, s128, u, We2, be2[None, :],
                       wg1_u, wg1_n, wg1_e, bg1[None, :], Wg2, bg2[None, :])
    return (x2T.T, e2T.T, u2)
